# baseline (speedup 1.0000x reference)
import functools

import jax
import jax.numpy as jnp
from jax import lax
from jax.experimental import pallas as pl
from jax.experimental.pallas import tpu as pltpu
from jax.experimental.pallas import tpu_sc as plsc

_REL = [(0, 1), (1, 0), (0, 1), (1, 0), (0, 0), (0, 0), (1, 1), (1, 1), (0, 0)]
NREL = 9
N = 50000
E = 500000
D = 128
NCH = 4
CW = 32
NS = 16
NA = 50048
RPT = NA // NS
BLK = 128
BPT = 248
NBLK = NS * BPT
EPAD = NBLK * BLK
NH = BPT // 2
NO = NH // 4


def _sc_agg_body(with_cnt, xall, src_off, dst2, zeros2d, zeros1d,
                 s_out, cnt_out, acc_sh, cnt_sh, srcb, dstb, rbuf, ones_v,
                 cbuf, sem_i, sem_g, sem_s, sem_c):
    core = lax.axis_index("c")
    t = lax.axis_index("s")
    r0 = t * RPT

    if with_cnt:
        for i in range(BLK // 16):
            ones_v[pl.ds(i * 16, 16)] = jnp.ones((16,), jnp.float32)

    def rel_body(r, _):
        _one_relation(with_cnt, r, core, t, r0, xall, src_off, dst2,
                      zeros2d, zeros1d, s_out, cnt_out, acc_sh, cnt_sh,
                      srcb, dstb, rbuf, ones_v, cbuf,
                      sem_i, sem_g, sem_s, sem_c)
        return None

    lax.fori_loop(0, NREL, rel_body, None)


def _one_relation(with_cnt, r, core, t, r0, xall, src_off, dst2,
                  zeros2d, zeros1d, s_out, cnt_out, acc_sh, cnt_sh,
                  srcb, dstb, rbuf, ones_v, cbuf,
                  sem_i, sem_g, sem_s, sem_c):
    for cc in range(2):
        chunk = core * 2 + cc

        pltpu.sync_copy(zeros2d, acc_sh.at[pl.ds(r0, RPT)])
        if with_cnt and cc == 0:
            @pl.when(core == 0)
            def _():
                pltpu.sync_copy(zeros1d.at[pl.ds(0, RPT)], cbuf)
                pltpu.sync_copy(cbuf, cnt_sh.at[pl.ds(r0, RPT)])
        plsc.subcore_barrier()

        def idx_fetch(h_sta, h_dyn):
            q = h_sta & 3
            row = t * BPT + h_dyn * 2
            pltpu.async_copy(src_off.at[r, chunk, pl.ds(row, 2)],
                             srcb.at[q], sem_i)
            pltpu.async_copy(dst2.at[r, pl.ds(row, 2)], dstb.at[q], sem_i)

        def wait_n(sem, n, mk):
            for _ in range(n):
                mk(sem).wait()

        mk_i = lambda s: pltpu.make_async_copy(dst2.at[0, pl.ds(0, 2)],
                                               dstb.at[0], s)
        mk_blk = lambda s: pltpu.make_async_copy(xall.at[pl.ds(0, BLK)],
                                                 rbuf.at[0], s)
        mk_cnt = lambda s: pltpu.make_async_copy(zeros1d.at[pl.ds(0, BLK)],
                                                 ones_v, s)

        idx_fetch(0, 0)
        idx_fetch(1, 1)

        def fire_scatters(hp, h_dyn):
            sp = (hp - 1) & 1
            q = (hp - 1) & 3
            for j in range(2):
                mk_blk(sem_g).wait()
                pltpu.async_copy(rbuf.at[sp * 2 + j],
                                 acc_sh.at[dstb.at[q, j]], sem_s, add=True)
                if with_cnt and cc == 0:
                    @pl.when(core == 0)
                    def _():
                        pltpu.async_copy(ones_v, cnt_sh.at[dstb.at[q, j]],
                                         sem_c, add=True)

        def outer_body(i, _):
            for hp in range(4):
                h = 4 * i + hp
                wait_n(sem_i, 2, mk_i)
                def free_slots():
                    wait_n(sem_s, 2, mk_blk)
                    if with_cnt and cc == 0:
                        @pl.when(core == 0)
                        def _():
                            wait_n(sem_c, 2, mk_cnt)
                if hp < 2:
                    @pl.when(i > 0)
                    def _():
                        free_slots()
                else:
                    free_slots()
                sp = hp & 1
                for j in range(2):
                    pltpu.async_copy(xall.at[srcb.at[hp & 3, j]],
                                     rbuf.at[sp * 2 + j], sem_g)
                if hp < 2:
                    idx_fetch(hp + 2, h + 2)
                else:
                    @pl.when(i < NO - 1)
                    def _():
                        idx_fetch(hp + 2, h + 2)
                if hp == 0:
                    @pl.when(i > 0)
                    def _():
                        fire_scatters(hp, h)
                else:
                    fire_scatters(hp, h)
            return None

        lax.fori_loop(0, NO, outer_body, None)

        fire_scatters(0, NH)
        wait_n(sem_s, 4, mk_blk)
        if with_cnt and cc == 0:
            @pl.when(core == 0)
            def _():
                wait_n(sem_c, 4, mk_cnt)

        plsc.subcore_barrier()
        pltpu.sync_copy(acc_sh.at[pl.ds(r0, RPT)],
                        s_out.at[r, pl.ds(r0, RPT), chunk])
        if with_cnt and cc == 0:
            @pl.when(core == 0)
            def _():
                pltpu.sync_copy(cnt_sh.at[pl.ds(r0, RPT)], cbuf)
                pltpu.sync_copy(cbuf, cnt_out.at[r, pl.ds(r0, RPT)])
        plsc.subcore_barrier()


def _make_sc_agg(with_cnt):
    mesh = plsc.VectorSubcoreMesh(core_axis_name="c", subcore_axis_name="s")
    out_type = [jax.ShapeDtypeStruct((NREL, NA, NCH, CW), jnp.float32),
                jax.ShapeDtypeStruct((NREL, NA), jnp.float32)]
    scratch = [
        pltpu.VMEM_SHARED((NA, CW), jnp.float32),
        pltpu.VMEM_SHARED((NA,), jnp.float32),
        pltpu.VMEM((4, 2, BLK), jnp.int32),
        pltpu.VMEM((4, 2, BLK), jnp.int32),
        pltpu.VMEM((4, BLK, CW), jnp.float32),
        pltpu.VMEM((BLK,), jnp.float32),
        pltpu.VMEM((RPT,), jnp.float32),
        pltpu.SemaphoreType.DMA,
        pltpu.SemaphoreType.DMA,
        pltpu.SemaphoreType.DMA,
        pltpu.SemaphoreType.DMA,
    ]
    return pl.kernel(functools.partial(_sc_agg_body, with_cnt),
                     out_type=out_type, mesh=mesh, scratch_types=scratch,
                     compiler_params=pltpu.CompilerParams(
                         use_tc_tiling_on_sc=False))


_sc_agg_cnt = _make_sc_agg(True)
_sc_agg_nocnt = _make_sc_agg(False)


def _prep_edges_all(eis):
    npad = EPAD - E
    pad_src = (jnp.arange(npad, dtype=jnp.int32) * 97) % N
    pad_dst = N + (jnp.arange(npad, dtype=jnp.int32) % (NA - N))
    srcp = jnp.stack([jnp.concatenate([ei[0], pad_src]) for ei in eis])
    dstp = jnp.stack([jnp.concatenate([ei[1], pad_dst]) for ei in eis])
    offs = jnp.asarray([[(tt * NCH + c) * N for c in range(NCH)]
                        for tt, _ in _REL], dtype=jnp.int32)
    src_off = (srcp[:, None, :] + offs[:, :, None]).reshape(
        NREL, NCH, NBLK, BLK)
    return src_off, dstp.reshape(NREL, NBLK, BLK)


def _chunk_major(x):
    return x.reshape(N, NCH, CW).transpose(1, 0, 2).reshape(NCH * N, CW)


def _tc_post_kernel(nrel, fuse_final, x_ref, *refs):
    s_refs = refs[:nrel]
    c_refs = refs[nrel:2 * nrel]
    wl_ref = refs[2 * nrel]
    wr_ref = refs[2 * nrel + 1]
    b_ref = refs[2 * nrel + 2]
    if fuse_final:
        wf_ref = refs[2 * nrel + 3]
        bf_ref = refs[2 * nrel + 4]
        o_ref = refs[2 * nrel + 5]
    else:
        o_ref = refs[2 * nrel + 3]
    acc = jnp.dot(x_ref[...], wr_ref[...],
                  preferred_element_type=jnp.float32) + b_ref[...]
    for r in range(nrel):
        inv = 1.0 / jnp.maximum(c_refs[r][...], 1.0)
        acc = acc + jnp.dot(s_refs[r][...] * inv, wl_ref[r],
                            preferred_element_type=jnp.float32)
    if fuse_final:
        o_ref[...] = jnp.maximum(
            jnp.dot(acc, wf_ref[...], preferred_element_type=jnp.float32)
            + bf_ref[...], 0.0)
    else:
        o_ref[...] = jnp.maximum(acc, 0.0)


def _tc_post(x_dst, s_list, cnt_list, Wl_stack, Wr_sum, b_sum, Wf=None, bf=None):
    nrel = len(s_list)
    fuse = Wf is not None
    nb = 50
    rblk = N // nb
    in_specs = [pl.BlockSpec((rblk, D), lambda i: (i, 0))]
    in_specs += [pl.BlockSpec((rblk, D), lambda i: (i, 0))] * nrel
    in_specs += [pl.BlockSpec((rblk, 1), lambda i: (i, 0))] * nrel
    in_specs += [pl.BlockSpec((nrel, D, D), lambda i: (0, 0, 0)),
                 pl.BlockSpec((D, D), lambda i: (0, 0)),
                 pl.BlockSpec((1, D), lambda i: (0, 0))]
    args = [x_dst] + s_list + cnt_list + [Wl_stack, Wr_sum, b_sum]
    if fuse:
        in_specs += [pl.BlockSpec((D, D), lambda i: (0, 0)),
                     pl.BlockSpec((1, D), lambda i: (0, 0))]
        args += [Wf, bf]
    return pl.pallas_call(
        functools.partial(_tc_post_kernel, nrel, fuse),
        grid=(nb,),
        in_specs=in_specs,
        out_specs=pl.BlockSpec((rblk, D), lambda i: (i, 0)),
        out_shape=jax.ShapeDtypeStruct((N, D), jnp.float32),
    )(*args)


def kernel(x_drug, x_disease, ei_treats, ei_treated_by, ei_contraind, ei_contraind_by,
           ei_drug_parent, ei_drug_child, ei_dis_parent, ei_dis_child, ei_interacts,
           W_l1, W_r1, b_l1, W_l2, W_r2, b_l2, W_drug, b_drug, W_dis, b_dis):
    eis = [ei_treats, ei_treated_by, ei_contraind, ei_contraind_by, ei_drug_parent,
           ei_drug_child, ei_dis_parent, ei_dis_child, ei_interacts]
    src_off, dstp = _prep_edges_all(eis)
    zeros2d = jnp.zeros((RPT, CW), jnp.float32)
    zeros1d = jnp.zeros((NA,), jnp.float32)

    rels_of = [[r for r, (s, dd) in enumerate(_REL) if dd == t] for t in (0, 1)]

    def run_layer(x0_flat, x1_flat, Wl, Wr, bl, cnts, first, Wf=None, bf=None):
        xall = jnp.concatenate([_chunk_major(x0_flat), _chunk_major(x1_flat)])
        if first:
            s_all, cnt_all = _sc_agg_cnt(xall, src_off, dstp, zeros2d, zeros1d)
            cnt_list = [cnt_all[r] for r in range(NREL)]
        else:
            s_all, _ = _sc_agg_nocnt(xall, src_off, dstp, zeros2d, zeros1d)
            cnt_list = cnts
        s_list = [s_all[r].reshape(NA, D) for r in range(NREL)]
        outs = []
        for t, x_dst in enumerate((x0_flat, x1_flat)):
            rs = rels_of[t]
            Wl_stack = jnp.stack([Wl[r] for r in rs])
            Wr_sum = sum(Wr[r] for r in rs)
            b_sum = sum(bl[r] for r in rs).reshape(1, D)
            wf = Wf[t] if Wf is not None else None
            bfr = bf[t].reshape(1, D) if bf is not None else None
            outs.append(_tc_post(
                x_dst, [s_list[r] for r in rs],
                [cnt_list[r][:N].reshape(N, 1) for r in rs],
                Wl_stack, Wr_sum, b_sum, wf, bfr))
        return outs[0], outs[1], cnt_list

    d1, s1, cnts = run_layer(x_drug, x_disease, W_l1, W_r1, b_l1, None, True)
    out_drug, out_dis, _ = run_layer(d1, s1, W_l2, W_r2, b_l2, cnts, False,
                                     Wf=(W_drug, W_dis), bf=(b_drug, b_dis))
    return out_drug, out_dis


# speedup vs baseline: 1.6781x; 1.6781x over previous
import functools

import jax
import jax.numpy as jnp
from jax import lax
from jax.experimental import pallas as pl
from jax.experimental.pallas import tpu as pltpu
from jax.experimental.pallas import tpu_sc as plsc

_REL = [(0, 1), (1, 0), (0, 1), (1, 0), (0, 0), (0, 0), (1, 1), (1, 1), (0, 0)]
NREL = 9
N = 50000
E = 500000
D = 128
NCH = 4
CW = 32
NS = 16
NA = 50048
RPT = NA // NS
BLK = 128
BPT = 248
NBLK = NS * BPT
EPAD = NBLK * BLK
NH = BPT // 2
NO = NH // 4


def _sc_agg_body(with_cnt, xv, src4, dst2, zeros2d, zeros1d,
                 s_out, cnt_out, acc_sh, cnt_sh, srcb, dstb, rbuf, ones_v,
                 cbuf, sem_i, sem_g, sem_s, sem_c):
    core = lax.axis_index("c")
    t = lax.axis_index("s")
    r0 = t * RPT

    if with_cnt:
        for i in range(BLK // 16):
            ones_v[pl.ds(i * 16, 16)] = jnp.ones((16,), jnp.float32)

    for cc in range(2):
        chunk = core * 2 + cc

        pltpu.sync_copy(zeros2d, acc_sh.at[pl.ds(r0, RPT)])
        if with_cnt and cc == 0:
            @pl.when(core == 0)
            def _():
                pltpu.sync_copy(zeros1d.at[pl.ds(0, RPT)], cbuf)
                pltpu.sync_copy(cbuf, cnt_sh.at[pl.ds(r0, RPT)])
        plsc.subcore_barrier()

        def idx_fetch(h_sta, h_dyn):
            q = h_sta & 3
            row = t * BPT + h_dyn * 2
            pltpu.async_copy(src4.at[pl.ds(row, 2)], srcb.at[q], sem_i)
            pltpu.async_copy(dst2.at[pl.ds(row, 2)], dstb.at[q], sem_i)

        def wait_n(sem, n, mk):
            for _ in range(n):
                mk(sem).wait()

        mk_i = lambda s: pltpu.make_async_copy(dst2.at[pl.ds(0, 2)],
                                               dstb.at[0], s)
        mk_blk = lambda s: pltpu.make_async_copy(xv.at[pl.ds(0, BLK)],
                                                 rbuf.at[0], s)
        mk_cnt = lambda s: pltpu.make_async_copy(zeros1d.at[pl.ds(0, BLK)],
                                                 ones_v, s)

        idx_fetch(0, 0)
        idx_fetch(1, 1)

        def fire_scatters(hp, h_dyn):
            sp = (hp - 1) & 1
            q = (hp - 1) & 3
            for j in range(2):
                mk_blk(sem_g).wait()
                pltpu.async_copy(rbuf.at[sp * 2 + j],
                                 acc_sh.at[dstb.at[q, j]], sem_s, add=True)
                if with_cnt and cc == 0:
                    @pl.when(core == 0)
                    def _():
                        pltpu.async_copy(ones_v, cnt_sh.at[dstb.at[q, j]],
                                         sem_c, add=True)

        def outer_body(i, _):
            for hp in range(4):
                h = 4 * i + hp
                wait_n(sem_i, 2, mk_i)
                for j in range(2):
                    for v in range(BLK // 16):
                        sl = pl.ds(v * 16, 16)
                        srcb[hp & 3, j, sl] = srcb[hp & 3, j, sl] + chunk
                def free_slots():
                    wait_n(sem_s, 2, mk_blk)
                    if with_cnt and cc == 0:
                        @pl.when(core == 0)
                        def _():
                            wait_n(sem_c, 2, mk_cnt)
                if hp < 2:
                    @pl.when(i > 0)
                    def _():
                        free_slots()
                else:
                    free_slots()
                sp = hp & 1
                for j in range(2):
                    pltpu.async_copy(xv.at[srcb.at[hp & 3, j]],
                                     rbuf.at[sp * 2 + j], sem_g)
                if hp < 2:
                    idx_fetch(hp + 2, h + 2)
                else:
                    @pl.when(i < NO - 1)
                    def _():
                        idx_fetch(hp + 2, h + 2)
                if hp == 0:
                    @pl.when(i > 0)
                    def _():
                        fire_scatters(hp, h)
                else:
                    fire_scatters(hp, h)
            return None

        lax.fori_loop(0, NO, outer_body, None)

        fire_scatters(0, NH)
        wait_n(sem_s, 4, mk_blk)
        if with_cnt and cc == 0:
            @pl.when(core == 0)
            def _():
                wait_n(sem_c, 4, mk_cnt)

        plsc.subcore_barrier()
        pltpu.sync_copy(acc_sh.at[pl.ds(r0, RPT)],
                        s_out.at[pl.ds(r0, RPT), chunk])
        if with_cnt and cc == 0:
            @pl.when(core == 0)
            def _():
                pltpu.sync_copy(cnt_sh.at[pl.ds(r0, RPT)], cbuf)
                pltpu.sync_copy(cbuf, cnt_out.at[pl.ds(r0, RPT)])
        plsc.subcore_barrier()


def _make_sc_agg(with_cnt):
    mesh = plsc.VectorSubcoreMesh(core_axis_name="c", subcore_axis_name="s")
    out_type = [jax.ShapeDtypeStruct((NA, NCH, CW), jnp.float32),
                jax.ShapeDtypeStruct((NA,), jnp.float32)]
    scratch = [
        pltpu.VMEM_SHARED((NA, CW), jnp.float32),
        pltpu.VMEM_SHARED((NA,), jnp.float32),
        pltpu.VMEM((4, 2, BLK), jnp.int32),
        pltpu.VMEM((4, 2, BLK), jnp.int32),
        pltpu.VMEM((4, BLK, CW), jnp.float32),
        pltpu.VMEM((BLK,), jnp.float32),
        pltpu.VMEM((RPT,), jnp.float32),
        pltpu.SemaphoreType.DMA,
        pltpu.SemaphoreType.DMA,
        pltpu.SemaphoreType.DMA,
        pltpu.SemaphoreType.DMA,
    ]
    return pl.kernel(functools.partial(_sc_agg_body, with_cnt),
                     out_type=out_type, mesh=mesh, scratch_types=scratch,
                     compiler_params=pltpu.CompilerParams(
                         use_tc_tiling_on_sc=False))


_sc_agg_cnt = _make_sc_agg(True)
_sc_agg_nocnt = _make_sc_agg(False)


def _prep_edges(ei):
    npad = EPAD - E
    pad_src = (jnp.arange(npad, dtype=jnp.int32) * 97) % N
    pad_dst = N + (jnp.arange(npad, dtype=jnp.int32) % (NA - N))
    src4 = (jnp.concatenate([ei[0], pad_src]) * NCH).reshape(NBLK, BLK)
    dstp = jnp.concatenate([ei[1], pad_dst]).reshape(NBLK, BLK)
    return src4, dstp


def _tc_post_kernel(nrel, fuse_final, x_ref, *refs):
    s_refs = refs[:nrel]
    c_refs = refs[nrel:2 * nrel]
    wl_ref = refs[2 * nrel]
    wr_ref = refs[2 * nrel + 1]
    b_ref = refs[2 * nrel + 2]
    if fuse_final:
        wf_ref = refs[2 * nrel + 3]
        bf_ref = refs[2 * nrel + 4]
        o_ref = refs[2 * nrel + 5]
    else:
        o_ref = refs[2 * nrel + 3]
    acc = jnp.dot(x_ref[...], wr_ref[...],
                  preferred_element_type=jnp.float32) + b_ref[...]
    for r in range(nrel):
        inv = 1.0 / jnp.maximum(c_refs[r][...], 1.0)
        acc = acc + jnp.dot(s_refs[r][...] * inv, wl_ref[r],
                            preferred_element_type=jnp.float32)
    if fuse_final:
        o_ref[...] = jnp.maximum(
            jnp.dot(acc, wf_ref[...], preferred_element_type=jnp.float32)
            + bf_ref[...], 0.0)
    else:
        o_ref[...] = jnp.maximum(acc, 0.0)


def _tc_post(x_dst, s_list, cnt_list, Wl_stack, Wr_sum, b_sum, Wf=None, bf=None):
    nrel = len(s_list)
    fuse = Wf is not None
    nb = 50
    rblk = N // nb
    in_specs = [pl.BlockSpec((rblk, D), lambda i: (i, 0))]
    in_specs += [pl.BlockSpec((rblk, D), lambda i: (i, 0))] * nrel
    in_specs += [pl.BlockSpec((rblk, 1), lambda i: (i, 0))] * nrel
    in_specs += [pl.BlockSpec((nrel, D, D), lambda i: (0, 0, 0)),
                 pl.BlockSpec((D, D), lambda i: (0, 0)),
                 pl.BlockSpec((1, D), lambda i: (0, 0))]
    args = [x_dst] + s_list + cnt_list + [Wl_stack, Wr_sum, b_sum]
    if fuse:
        in_specs += [pl.BlockSpec((D, D), lambda i: (0, 0)),
                     pl.BlockSpec((1, D), lambda i: (0, 0))]
        args += [Wf, bf]
    return pl.pallas_call(
        functools.partial(_tc_post_kernel, nrel, fuse),
        grid=(nb,),
        in_specs=in_specs,
        out_specs=pl.BlockSpec((rblk, D), lambda i: (i, 0)),
        out_shape=jax.ShapeDtypeStruct((N, D), jnp.float32),
    )(*args)


def kernel(x_drug, x_disease, ei_treats, ei_treated_by, ei_contraind, ei_contraind_by,
           ei_drug_parent, ei_drug_child, ei_dis_parent, ei_dis_child, ei_interacts,
           W_l1, W_r1, b_l1, W_l2, W_r2, b_l2, W_drug, b_drug, W_dis, b_dis):
    eis = [ei_treats, ei_treated_by, ei_contraind, ei_contraind_by, ei_drug_parent,
           ei_drug_child, ei_dis_parent, ei_dis_child, ei_interacts]
    edges = [_prep_edges(ei) for ei in eis]
    zeros2d = jnp.zeros((RPT, CW), jnp.float32)
    zeros1d = jnp.zeros((NA,), jnp.float32)

    rels_of = [[r for r, (s, dd) in enumerate(_REL) if dd == t] for t in (0, 1)]

    def run_layer(x0_flat, x1_flat, Wl, Wr, bl, cnts, first, Wf=None, bf=None):
        tables = [x0_flat.reshape(N * NCH, CW), x1_flat.reshape(N * NCH, CW)]
        s_list, cnt_list = [], []
        for r, (st, _) in enumerate(_REL):
            src4, dstp = edges[r]
            if first:
                s, c = _sc_agg_cnt(tables[st], src4, dstp, zeros2d, zeros1d)
                cnt_list.append(c)
            else:
                s, _ = _sc_agg_nocnt(tables[st], src4, dstp, zeros2d, zeros1d)
                cnt_list.append(cnts[r])
            s_list.append(s.reshape(NA, D))
        outs = []
        for t, x_dst in enumerate((x0_flat, x1_flat)):
            rs = rels_of[t]
            Wl_stack = jnp.stack([Wl[r] for r in rs])
            Wr_sum = sum(Wr[r] for r in rs)
            b_sum = sum(bl[r] for r in rs).reshape(1, D)
            wf = Wf[t] if Wf is not None else None
            bfr = bf[t].reshape(1, D) if bf is not None else None
            outs.append(_tc_post(
                x_dst, [s_list[r] for r in rs],
                [cnt_list[r][:N].reshape(N, 1) for r in rs],
                Wl_stack, Wr_sum, b_sum, wf, bfr))
        return outs[0], outs[1], cnt_list

    d1, s1, cnts = run_layer(x_drug, x_disease, W_l1, W_r1, b_l1, None, True)
    out_drug, out_dis, _ = run_layer(d1, s1, W_l2, W_r2, b_l2, cnts, False,
                                     Wf=(W_drug, W_dis), bf=(b_drug, b_dis))
    return out_drug, out_dis


# speedup vs baseline: 1.7074x; 1.0175x over previous
import functools

import jax
import jax.numpy as jnp
from jax import lax
from jax.experimental import pallas as pl
from jax.experimental.pallas import tpu as pltpu
from jax.experimental.pallas import tpu_sc as plsc

_REL = [(0, 1), (1, 0), (0, 1), (1, 0), (0, 0), (0, 0), (1, 1), (1, 1), (0, 0)]
NREL = 9
N = 50000
E = 500000
D = 128
NCH = 4
CW = 32
NS = 16
NA = 50048
RPT = NA // NS
BLK = 128
BPT = 248
NBLK = NS * BPT
EPAD = NBLK * BLK
NH = BPT // 2
NO = NH // 4


def _sc_agg_body(with_cnt, xv, eip, zeros2d, zeros1d,
                 s_out, cnt_out, acc_sh, cnt_sh, ib, rbuf, ones_v,
                 cbuf, sem_i, sem_g, sem_s, sem_c):
    core = lax.axis_index("c")
    t = lax.axis_index("s")
    r0 = t * RPT

    if with_cnt:
        for i in range(BLK // 16):
            ones_v[pl.ds(i * 16, 16)] = jnp.ones((16,), jnp.float32)

    for cc in range(2):
        chunk = core * 2 + cc

        pltpu.sync_copy(zeros2d, acc_sh.at[pl.ds(r0, RPT)])
        if with_cnt and cc == 0:
            @pl.when(core == 0)
            def _():
                pltpu.sync_copy(zeros1d.at[pl.ds(0, RPT)], cbuf)
                pltpu.sync_copy(cbuf, cnt_sh.at[pl.ds(r0, RPT)])
        plsc.subcore_barrier()

        def idx_fetch(h_sta, h_dyn):
            q = h_sta & 3
            row = t * BPT + h_dyn * 2
            pltpu.async_copy(eip.at[pl.ds(row, 2)], ib.at[q], sem_i)

        def wait_n(sem, n, mk):
            for _ in range(n):
                mk(sem).wait()

        mk_i = lambda s: pltpu.make_async_copy(eip.at[pl.ds(0, 2)],
                                               ib.at[0], s)
        mk_blk = lambda s: pltpu.make_async_copy(xv.at[pl.ds(0, BLK)],
                                                 rbuf.at[0], s)
        mk_cnt = lambda s: pltpu.make_async_copy(zeros1d.at[pl.ds(0, BLK)],
                                                 ones_v, s)

        idx_fetch(0, 0)
        idx_fetch(1, 1)

        def fire_scatters(hp, h_dyn):
            sp = (hp - 1) & 1
            q = (hp - 1) & 3
            for j in range(2):
                mk_blk(sem_g).wait()
                pltpu.async_copy(rbuf.at[sp * 2 + j],
                                 acc_sh.at[ib.at[q, j, 1]], sem_s, add=True)
                if with_cnt and cc == 0:
                    @pl.when(core == 0)
                    def _():
                        pltpu.async_copy(ones_v, cnt_sh.at[ib.at[q, j, 1]],
                                         sem_c, add=True)

        def outer_body(i, _):
            for hp in range(4):
                h = 4 * i + hp
                wait_n(sem_i, 1, mk_i)
                for j in range(2):
                    for v in range(BLK // 16):
                        sl = pl.ds(v * 16, 16)
                        ib[hp & 3, j, 0, sl] = ib[hp & 3, j, 0, sl] + chunk
                def free_slots():
                    wait_n(sem_s, 2, mk_blk)
                    if with_cnt and cc == 0:
                        @pl.when(core == 0)
                        def _():
                            wait_n(sem_c, 2, mk_cnt)
                if hp < 2:
                    @pl.when(i > 0)
                    def _():
                        free_slots()
                else:
                    free_slots()
                sp = hp & 1
                for j in range(2):
                    pltpu.async_copy(xv.at[ib.at[hp & 3, j, 0]],
                                     rbuf.at[sp * 2 + j], sem_g)
                if hp < 2:
                    idx_fetch(hp + 2, h + 2)
                else:
                    @pl.when(i < NO - 1)
                    def _():
                        idx_fetch(hp + 2, h + 2)
                if hp == 0:
                    @pl.when(i > 0)
                    def _():
                        fire_scatters(hp, h)
                else:
                    fire_scatters(hp, h)
            return None

        lax.fori_loop(0, NO, outer_body, None)

        fire_scatters(0, NH)
        wait_n(sem_s, 4, mk_blk)
        if with_cnt and cc == 0:
            @pl.when(core == 0)
            def _():
                wait_n(sem_c, 4, mk_cnt)

        plsc.subcore_barrier()
        pltpu.sync_copy(acc_sh.at[pl.ds(r0, RPT)],
                        s_out.at[pl.ds(r0, RPT), chunk])
        if with_cnt and cc == 0:
            @pl.when(core == 0)
            def _():
                pltpu.sync_copy(cnt_sh.at[pl.ds(r0, RPT)], cbuf)
                pltpu.sync_copy(cbuf, cnt_out.at[pl.ds(r0, RPT)])
        plsc.subcore_barrier()


def _make_sc_agg(with_cnt):
    mesh = plsc.VectorSubcoreMesh(core_axis_name="c", subcore_axis_name="s")
    out_type = [jax.ShapeDtypeStruct((NA, NCH, CW), jnp.float32),
                jax.ShapeDtypeStruct((NA,), jnp.float32)]
    scratch = [
        pltpu.VMEM_SHARED((NA, CW), jnp.float32),
        pltpu.VMEM_SHARED((NA,), jnp.float32),
        pltpu.VMEM((4, 2, 2, BLK), jnp.int32),
        pltpu.VMEM((4, BLK, CW), jnp.float32),
        pltpu.VMEM((BLK,), jnp.float32),
        pltpu.VMEM((RPT,), jnp.float32),
        pltpu.SemaphoreType.DMA,
        pltpu.SemaphoreType.DMA,
        pltpu.SemaphoreType.DMA,
        pltpu.SemaphoreType.DMA,
    ]
    return pl.kernel(functools.partial(_sc_agg_body, with_cnt),
                     out_type=out_type, mesh=mesh, scratch_types=scratch,
                     compiler_params=pltpu.CompilerParams(
                         use_tc_tiling_on_sc=False))


_sc_agg_cnt = _make_sc_agg(True)
_sc_agg_nocnt = _make_sc_agg(False)


def _prep_edges(ei):
    npad = EPAD - E
    pad_src = (jnp.arange(npad, dtype=jnp.int32) * 97) % N
    pad_dst = N + (jnp.arange(npad, dtype=jnp.int32) % (NA - N))
    src4 = (jnp.concatenate([ei[0], pad_src]) * NCH).reshape(NBLK, BLK)
    dstp = jnp.concatenate([ei[1], pad_dst]).reshape(NBLK, BLK)
    return jnp.stack([src4, dstp], axis=1)


def _tc_post_kernel(nrel, fuse_final, x_ref, *refs):
    s_refs = refs[:nrel]
    c_refs = refs[nrel:2 * nrel]
    wl_ref = refs[2 * nrel]
    wr_ref = refs[2 * nrel + 1]
    b_ref = refs[2 * nrel + 2]
    if fuse_final:
        wf_ref = refs[2 * nrel + 3]
        bf_ref = refs[2 * nrel + 4]
        o_ref = refs[2 * nrel + 5]
    else:
        o_ref = refs[2 * nrel + 3]
    acc = jnp.dot(x_ref[...], wr_ref[...],
                  preferred_element_type=jnp.float32) + b_ref[...]
    for r in range(nrel):
        inv = 1.0 / jnp.maximum(c_refs[r][...], 1.0)
        acc = acc + jnp.dot(s_refs[r][...] * inv, wl_ref[r],
                            preferred_element_type=jnp.float32)
    if fuse_final:
        o_ref[...] = jnp.maximum(
            jnp.dot(acc, wf_ref[...], preferred_element_type=jnp.float32)
            + bf_ref[...], 0.0)
    else:
        o_ref[...] = jnp.maximum(acc, 0.0)


def _tc_post(x_dst, s_list, cnt_list, Wl_stack, Wr_sum, b_sum, Wf=None, bf=None):
    nrel = len(s_list)
    fuse = Wf is not None
    nb = 50
    rblk = N // nb
    in_specs = [pl.BlockSpec((rblk, D), lambda i: (i, 0))]
    in_specs += [pl.BlockSpec((rblk, D), lambda i: (i, 0))] * nrel
    in_specs += [pl.BlockSpec((rblk, 1), lambda i: (i, 0))] * nrel
    in_specs += [pl.BlockSpec((nrel, D, D), lambda i: (0, 0, 0)),
                 pl.BlockSpec((D, D), lambda i: (0, 0)),
                 pl.BlockSpec((1, D), lambda i: (0, 0))]
    args = [x_dst] + s_list + cnt_list + [Wl_stack, Wr_sum, b_sum]
    if fuse:
        in_specs += [pl.BlockSpec((D, D), lambda i: (0, 0)),
                     pl.BlockSpec((1, D), lambda i: (0, 0))]
        args += [Wf, bf]
    return pl.pallas_call(
        functools.partial(_tc_post_kernel, nrel, fuse),
        grid=(nb,),
        in_specs=in_specs,
        out_specs=pl.BlockSpec((rblk, D), lambda i: (i, 0)),
        out_shape=jax.ShapeDtypeStruct((N, D), jnp.float32),
    )(*args)


def kernel(x_drug, x_disease, ei_treats, ei_treated_by, ei_contraind, ei_contraind_by,
           ei_drug_parent, ei_drug_child, ei_dis_parent, ei_dis_child, ei_interacts,
           W_l1, W_r1, b_l1, W_l2, W_r2, b_l2, W_drug, b_drug, W_dis, b_dis):
    eis = [ei_treats, ei_treated_by, ei_contraind, ei_contraind_by, ei_drug_parent,
           ei_drug_child, ei_dis_parent, ei_dis_child, ei_interacts]
    edges = [_prep_edges(ei) for ei in eis]
    zeros2d = jnp.zeros((RPT, CW), jnp.float32)
    zeros1d = jnp.zeros((NA,), jnp.float32)

    rels_of = [[r for r, (s, dd) in enumerate(_REL) if dd == t] for t in (0, 1)]

    def run_layer(x0_flat, x1_flat, Wl, Wr, bl, cnts, first, Wf=None, bf=None):
        tables = [x0_flat.reshape(N * NCH, CW), x1_flat.reshape(N * NCH, CW)]
        s_list, cnt_list = [], []
        for r, (st, _) in enumerate(_REL):
            if first:
                s, c = _sc_agg_cnt(tables[st], edges[r], zeros2d, zeros1d)
                cnt_list.append(c)
            else:
                s, _ = _sc_agg_nocnt(tables[st], edges[r], zeros2d, zeros1d)
                cnt_list.append(cnts[r])
            s_list.append(s.reshape(NA, D))
        outs = []
        for t, x_dst in enumerate((x0_flat, x1_flat)):
            rs = rels_of[t]
            Wl_stack = jnp.stack([Wl[r] for r in rs])
            Wr_sum = sum(Wr[r] for r in rs)
            b_sum = sum(bl[r] for r in rs).reshape(1, D)
            wf = Wf[t] if Wf is not None else None
            bfr = bf[t].reshape(1, D) if bf is not None else None
            outs.append(_tc_post(
                x_dst, [s_list[r] for r in rs],
                [cnt_list[r][:N].reshape(N, 1) for r in rs],
                Wl_stack, Wr_sum, b_sum, wf, bfr))
        return outs[0], outs[1], cnt_list

    d1, s1, cnts = run_layer(x_drug, x_disease, W_l1, W_r1, b_l1, None, True)
    out_drug, out_dis, _ = run_layer(d1, s1, W_l2, W_r2, b_l2, cnts, False,
                                     Wf=(W_drug, W_dis), bf=(b_drug, b_dis))
    return out_drug, out_dis
